# baseline (speedup 1.0000x reference)
import functools

import jax
import jax.numpy as jnp
from jax import lax
from jax.experimental import pallas as pl
from jax.experimental.pallas import tpu as pltpu
from jax.experimental.pallas import tpu_sc as plsc

D = 64
ROWS = 4096
TOK = 200
NC, NS, LANES = 2, 16, 16
NW = NC * NS
R_PER_W = ROWS // NW
BLK = 8
NBLK = TOK // BLK
SCALE = 8.0

_mesh = plsc.VectorSubcoreMesh(
    core_axis_name="c", subcore_axis_name="s", num_cores=NC, num_subcores=NS
)

_scratch = (
    [pltpu.VMEM((BLK, R_PER_W), jnp.int32) for _ in range(2)]
    + [pltpu.VMEM((BLK, R_PER_W), jnp.int32) for _ in range(2)]
    + [pltpu.VMEM((BLK, R_PER_W), jnp.int32) for _ in range(2)]
    + [pltpu.VMEM((R_PER_W, 128), jnp.float32) for _ in range(2)]
    + [pltpu.VMEM((1, D, R_PER_W), jnp.float32) for _ in range(2)]
    + [pltpu.SemaphoreType.DMA for _ in range(6)]
)


@functools.partial(
    pl.kernel,
    out_type=jax.ShapeDtypeStruct((TOK, D, ROWS), jnp.float32),
    mesh=_mesh,
    scratch_types=_scratch,
    compiler_params=pltpu.CompilerParams(
        use_tc_tiling_on_sc=True, needs_layout_passes=False
    ),
)
def _emb_kernel(xt_hbm, table_hbm, out_hbm, i0, i1, s0, s1, p0, p1,
                g0, g1, o0, o1, is0, is1, gs0, gs1, ws0, ws1):
    ibuf, sbuf, pbuf = (i0, i1), (s0, s1), (p0, p1)
    gbuf, obuf = (g0, g1), (o0, o1)
    isem, gsem, wsem = (is0, is1), (gs0, gs1), (ws0, ws1)

    wid = lax.axis_index("s") * NC + lax.axis_index("c")
    r0 = wid * R_PER_W

    def ifetch(sb, blk):
        pltpu.async_copy(
            xt_hbm.at[pl.ds(blk * BLK, BLK), pl.ds(r0, R_PER_W)],
            ibuf[sb], isem[sb],
        )

    def iwait(sb, blk):
        pltpu.make_async_copy(
            xt_hbm.at[pl.ds(blk * BLK, BLK), pl.ds(r0, R_PER_W)],
            ibuf[sb], isem[sb],
        ).wait()

    def prep(sb):
        src, dst_s, dst_p = ibuf[sb], sbuf[sb], pbuf[sb]

        @plsc.parallel_loop(0, BLK * (R_PER_W // LANES), unroll=4)
        def _(k):
            i = k // (R_PER_W // LANES)
            sl = pl.ds((k % (R_PER_W // LANES)) * LANES, LANES)
            v = src[i, sl]
            dst_s[i, sl] = lax.shift_right_logical(v, 1)
            dst_p[i, sl] = lax.shift_left(v & 1, 6)

    def start_gather(s2, sb, i):
        pltpu.async_copy(table_hbm.at[sbuf[sb].at[i]], gbuf[s2], gsem[s2])

    def wait_gather(s2, sb, i):
        pltpu.make_async_copy(
            table_hbm.at[sbuf[sb].at[i]], gbuf[s2], gsem[s2]
        ).wait()

    def start_write(s2, t):
        pltpu.async_copy(
            obuf[s2],
            out_hbm.at[pl.ds(t, 1), pl.ds(0, D), pl.ds(r0, R_PER_W)],
            wsem[s2],
        )

    def wait_write(s2, t):
        pltpu.make_async_copy(
            obuf[s2],
            out_hbm.at[pl.ds(t, 1), pl.ds(0, D), pl.ds(r0, R_PER_W)],
            wsem[s2],
        ).wait()

    def transpose_scale(s2, sb, i):
        src, dst, par = gbuf[s2], obuf[s2], pbuf[sb]
        for j in range(R_PER_W // LANES):
            rows = j * LANES + lax.iota(jnp.int32, 16)
            parv = par[i, pl.ds(j * LANES, LANES)]

            @plsc.parallel_loop(0, D, unroll=4)
            def _(d):
                cols = parv + d
                dst[0, d, pl.ds(j * LANES, LANES)] = (
                    plsc.load_gather(src, [rows, cols]) * SCALE
                )

    ifetch(0, 0)
    iwait(0, 0)
    prep(0)
    ifetch(1, 1)
    start_gather(0, 0, 0)
    start_gather(1, 0, 1)

    def process_block(blk, sb, last):
        for i in range(BLK):
            t = blk * BLK + i
            s2 = i % 2
            wait_gather(s2, sb, i)

            if last:
                if t >= 2:
                    wait_write(s2, t)
            else:
                @pl.when(t > 1)
                def _():
                    wait_write(s2, t)

            transpose_scale(s2, sb, i)
            start_write(s2, t)

            if i < BLK - 2:
                if i == 5 and not last:
                    iwait(1 - sb, blk + 1)
                    prep(1 - sb)

                    @pl.when(blk + 2 < NBLK)
                    def _():
                        ifetch(sb, blk + 2)
                start_gather(s2, sb, i + 2)
            elif not last:
                start_gather(s2, 1 - sb, i + 2 - BLK)

    @pl.loop(0, (NBLK - 1) // 2)
    def _(t2):
        for p in range(2):
            process_block(t2 * 2 + p, p, last=False)

    process_block(NBLK - 1, 0, last=True)

    wait_write(0, TOK - 2)
    wait_write(1, TOK - 1)


def kernel(x, table):
    xt = x.astype(jnp.int32).T
    t128 = table.reshape(500000, 128)
    out_t = _emb_kernel(xt, t128)
    return out_t.transpose(2, 0, 1)


# speedup vs baseline: 1.0103x; 1.0103x over previous
import functools

import jax
import jax.numpy as jnp
from jax import lax
from jax.experimental import pallas as pl
from jax.experimental.pallas import tpu as pltpu
from jax.experimental.pallas import tpu_sc as plsc

D = 64
ROWS = 4096
TOK = 200
NC, NS, LANES = 2, 16, 16
NW = NC * NS
R_PER_W = ROWS // NW
BLK = 8
NBLK = TOK // BLK
NSLOT = 4
SCALE = 8.0

_mesh = plsc.VectorSubcoreMesh(
    core_axis_name="c", subcore_axis_name="s", num_cores=NC, num_subcores=NS
)

_scratch = (
    [pltpu.VMEM((BLK, R_PER_W), jnp.int32) for _ in range(2)]
    + [pltpu.VMEM((BLK, R_PER_W), jnp.int32) for _ in range(2)]
    + [pltpu.VMEM((BLK, R_PER_W), jnp.int32) for _ in range(2)]
    + [pltpu.VMEM((R_PER_W, 128), jnp.float32) for _ in range(NSLOT)]
    + [pltpu.VMEM((1, D, R_PER_W), jnp.float32) for _ in range(NSLOT)]
    + [pltpu.SemaphoreType.DMA for _ in range(2 + 2 * NSLOT)]
)


@functools.partial(
    pl.kernel,
    out_type=jax.ShapeDtypeStruct((TOK, D, ROWS), jnp.float32),
    mesh=_mesh,
    scratch_types=_scratch,
    compiler_params=pltpu.CompilerParams(
        use_tc_tiling_on_sc=True, needs_layout_passes=False
    ),
)
def _emb_kernel(xt_hbm, table_hbm, out_hbm, *refs):
    ibuf = refs[0:2]
    sbuf = refs[2:4]
    pbuf = refs[4:6]
    gbuf = refs[6:6 + NSLOT]
    obuf = refs[6 + NSLOT:6 + 2 * NSLOT]
    isem = refs[6 + 2 * NSLOT:8 + 2 * NSLOT]
    gsem = refs[8 + 2 * NSLOT:8 + 3 * NSLOT]
    wsem = refs[8 + 3 * NSLOT:8 + 4 * NSLOT]

    wid = lax.axis_index("s") * NC + lax.axis_index("c")
    r0 = wid * R_PER_W

    def ifetch(sb, blk):
        pltpu.async_copy(
            xt_hbm.at[pl.ds(blk * BLK, BLK), pl.ds(r0, R_PER_W)],
            ibuf[sb], isem[sb],
        )

    def iwait(sb, blk):
        pltpu.make_async_copy(
            xt_hbm.at[pl.ds(blk * BLK, BLK), pl.ds(r0, R_PER_W)],
            ibuf[sb], isem[sb],
        ).wait()

    def prep(sb):
        src, dst_s, dst_p = ibuf[sb], sbuf[sb], pbuf[sb]
        nv = R_PER_W // LANES

        @plsc.parallel_loop(0, BLK * nv, unroll=4)
        def _(k):
            i = k // nv
            sl = pl.ds((k % nv) * LANES, LANES)
            v = src[i, sl]
            dst_s[i, sl] = lax.shift_right_logical(v, 1)
            dst_p[i, sl] = lax.shift_left(v & 1, 6)

    def start_gather(s, sb, i):
        pltpu.async_copy(table_hbm.at[sbuf[sb].at[i]], gbuf[s], gsem[s])

    def wait_gather(s, sb, i):
        pltpu.make_async_copy(
            table_hbm.at[sbuf[sb].at[i]], gbuf[s], gsem[s]
        ).wait()

    def start_write(s, t):
        pltpu.async_copy(
            obuf[s],
            out_hbm.at[pl.ds(t, 1), pl.ds(0, D), pl.ds(r0, R_PER_W)],
            wsem[s],
        )

    def wait_write(s, t):
        pltpu.make_async_copy(
            obuf[s],
            out_hbm.at[pl.ds(t, 1), pl.ds(0, D), pl.ds(r0, R_PER_W)],
            wsem[s],
        ).wait()

    def transpose_scale(s, sb, i):
        src, dst, par = gbuf[s], obuf[s], pbuf[sb]
        for j in range(R_PER_W // LANES):
            rows = j * LANES + lax.iota(jnp.int32, 16)
            parv = par[i, pl.ds(j * LANES, LANES)]

            @plsc.parallel_loop(0, D, unroll=4)
            def _(d):
                cols = parv + d
                dst[0, d, pl.ds(j * LANES, LANES)] = (
                    plsc.load_gather(src, [rows, cols]) * SCALE
                )

    ifetch(0, 0)
    iwait(0, 0)
    prep(0)
    ifetch(1, 1)
    for s in range(NSLOT):
        start_gather(s, 0, s)

    def process_block(blk, sb, last):
        for i in range(BLK):
            t = blk * BLK + i
            s = i % NSLOT
            wait_gather(s, sb, i)

            if last:
                if t >= NSLOT:
                    wait_write(s, t)
            else:
                @pl.when(t > NSLOT - 1)
                def _():
                    wait_write(s, t)

            if i == 3 and not last:
                iwait(1 - sb, blk + 1)
                prep(1 - sb)

                @pl.when(blk + 2 < NBLK)
                def _():
                    ifetch(sb, blk + 2)

            transpose_scale(s, sb, i)
            start_write(s, t)

            if i < BLK - NSLOT:
                start_gather(s, sb, i + NSLOT)
            elif not last:
                start_gather(s, 1 - sb, i + NSLOT - BLK)

    @pl.loop(0, (NBLK - 1) // 2)
    def _(t2):
        for p in range(2):
            process_block(t2 * 2 + p, p, last=False)

    process_block(NBLK - 1, 0, last=True)

    for s in range(NSLOT):
        wait_write(s, TOK - NSLOT + s)


def kernel(x, table):
    xt = x.astype(jnp.int32).T
    t128 = table.reshape(500000, 128)
    out_t = _emb_kernel(xt, t128)
    return out_t.transpose(2, 0, 1)
